# baseline (speedup 1.0000x reference)
import jax
import jax.numpy as jnp
from jax.experimental import pallas as pl


def _probe_body(act_ref, w_ref, o_ref):
    o_ref[...] = act_ref[...] + jnp.sum(w_ref[...])


def kernel(entity_hiddens, encoded_question, keys_mask, H, W_out, b_out):
    B, N, D = entity_hiddens.shape
    V = W_out.shape[1]
    wv = W_out.reshape(D * V // 128, 128)
    out = pl.pallas_call(
        _probe_body,
        grid=(1,),
        in_specs=[
            pl.BlockSpec((B, D), lambda j: (0, 0)),
            pl.BlockSpec((8, 128), lambda j: (0, 0)),
        ],
        out_specs=pl.BlockSpec((B, D), lambda j: (0, 0)),
        out_shape=jax.ShapeDtypeStruct((B, D), jnp.float32),
    )(encoded_question, wv)
    return jnp.pad(out[:, :1], ((0, 0), (0, V - 1)))


# speedup vs baseline: 1.4462x; 1.4462x over previous
import jax
import jax.numpy as jnp
from jax.experimental import pallas as pl

_TB = 4
_TV = 2048


def _attn_act_body(q_ref, m_ref, e_ref, h_ref, act_ref):
    e = e_ref[...]
    q = q_ref[0]
    m = m_ref[0]
    e16 = e.astype(jnp.bfloat16).astype(jnp.float32)
    q16 = q.astype(jnp.bfloat16).astype(jnp.float32)
    logits = jnp.sum(e16 * q16[:, None, :], axis=-1)
    logits = jnp.where(m > 0.0, logits, jnp.float32(-20.0))
    logits = logits - jnp.max(logits, axis=-1, keepdims=True)
    p = jnp.exp(logits)
    coef = p / jnp.sum(p, axis=-1, keepdims=True)
    u = jnp.sum(coef[:, :, None] * e, axis=1)
    z = q + jax.lax.dot_general(u, h_ref[...], (((1,), (0,)), ((), ())),
                                preferred_element_type=jnp.float32)
    act_ref[0] = 1.0 / (1.0 + jnp.exp(-z))


def _proj_body(act_ref, w_ref, b_ref, o_ref):
    o_ref[...] = jax.lax.dot_general(
        act_ref[...], w_ref[...], (((1,), (0,)), ((), ())),
        preferred_element_type=jnp.float32) + b_ref[...]


def kernel(entity_hiddens, encoded_question, keys_mask, H, W_out, b_out):
    B, N, D = entity_hiddens.shape
    V = W_out.shape[1]
    tb = _TB if B % _TB == 0 else 1
    tv = min(_TV, V)
    mask_f = keys_mask.astype(jnp.float32)

    nb = B // tb
    q3 = encoded_question.reshape(nb, tb, D)
    m3 = mask_f.reshape(nb, tb, N)
    act = pl.pallas_call(
        _attn_act_body,
        grid=(nb,),
        in_specs=[
            pl.BlockSpec((1, tb, D), lambda i: (i, 0, 0)),
            pl.BlockSpec((1, tb, N), lambda i: (i, 0, 0)),
            pl.BlockSpec((tb, N, D), lambda i: (i, 0, 0)),
            pl.BlockSpec((D, D), lambda i: (0, 0)),
        ],
        out_specs=pl.BlockSpec((1, tb, D), lambda i: (i, 0, 0)),
        out_shape=jax.ShapeDtypeStruct((nb, tb, D), jnp.float32),
    )(q3, m3, entity_hiddens, H)
    act = act.reshape(B, D)

    b2 = b_out.reshape(1, V)
    out = pl.pallas_call(
        _proj_body,
        grid=(pl.cdiv(V, tv),),
        in_specs=[
            pl.BlockSpec((B, D), lambda j: (0, 0)),
            pl.BlockSpec((D, tv), lambda j: (0, j)),
            pl.BlockSpec((1, tv), lambda j: (0, j)),
        ],
        out_specs=pl.BlockSpec((B, tv), lambda j: (0, j)),
        out_shape=jax.ShapeDtypeStruct((B, V), jnp.float32),
    )(act, W_out, b2)
    return out


# speedup vs baseline: 1.4766x; 1.0211x over previous
import jax
import jax.numpy as jnp
from jax.experimental import pallas as pl

_TB = 4
_TV = 2048


def _attn_body(q_ref, m_ref, e_ref, u_ref):
    e = e_ref[...]
    q = q_ref[0]
    m = m_ref[0]
    e16 = e.astype(jnp.bfloat16).astype(jnp.float32)
    q16 = q.astype(jnp.bfloat16).astype(jnp.float32)
    logits = jnp.sum(e16 * q16[:, None, :], axis=-1)
    logits = jnp.where(m > 0.0, logits, jnp.float32(-20.0))
    logits = logits - jnp.max(logits, axis=-1, keepdims=True)
    p = jnp.exp(logits)
    coef = p / jnp.sum(p, axis=-1, keepdims=True)
    u_ref[0] = jnp.sum(coef[:, :, None] * e, axis=1)


def _act_body(q_ref, u_ref, h_ref, act_ref):
    z = q_ref[...] + jax.lax.dot_general(
        u_ref[...], h_ref[...], (((1,), (0,)), ((), ())),
        preferred_element_type=jnp.float32)
    act_ref[...] = 1.0 / (1.0 + jnp.exp(-z))


def _proj_body(act_ref, w_ref, b_ref, o_ref):
    o_ref[...] = jax.lax.dot_general(
        act_ref[...], w_ref[...], (((1,), (0,)), ((), ())),
        preferred_element_type=jnp.float32) + b_ref[...]


def kernel(entity_hiddens, encoded_question, keys_mask, H, W_out, b_out):
    B, N, D = entity_hiddens.shape
    V = W_out.shape[1]
    tb = _TB if B % _TB == 0 else 1
    tv = min(_TV, V)
    mask_f = keys_mask.astype(jnp.float32)

    nb = B // tb
    q3 = encoded_question.reshape(nb, tb, D)
    m3 = mask_f.reshape(nb, tb, N)
    u = pl.pallas_call(
        _attn_body,
        grid=(nb,),
        in_specs=[
            pl.BlockSpec((1, tb, D), lambda i: (i, 0, 0)),
            pl.BlockSpec((1, tb, N), lambda i: (i, 0, 0)),
            pl.BlockSpec((tb, N, D), lambda i: (i, 0, 0)),
        ],
        out_specs=pl.BlockSpec((1, tb, D), lambda i: (i, 0, 0)),
        out_shape=jax.ShapeDtypeStruct((nb, tb, D), jnp.float32),
    )(q3, m3, entity_hiddens)
    u = u.reshape(B, D)

    act = pl.pallas_call(
        _act_body,
        out_shape=jax.ShapeDtypeStruct((B, D), jnp.float32),
    )(encoded_question, u, H)

    b2 = b_out.reshape(1, V)
    out = pl.pallas_call(
        _proj_body,
        grid=(pl.cdiv(V, tv),),
        in_specs=[
            pl.BlockSpec((B, D), lambda j: (0, 0)),
            pl.BlockSpec((D, tv), lambda j: (0, j)),
            pl.BlockSpec((1, tv), lambda j: (0, j)),
        ],
        out_specs=pl.BlockSpec((B, tv), lambda j: (0, j)),
        out_shape=jax.ShapeDtypeStruct((B, V), jnp.float32),
    )(act, W_out, b2)
    return out
